# Initial kernel scaffold; baseline (speedup 1.0000x reference)
import jax
import jax.numpy as jnp
from jax.experimental import pallas as pl


def kernel(x, enc0_Wg, enc0_bg, enc0_Wc, enc0_bc, enc1_Wg, enc1_bg, enc1_Wc, enc1_bc, dec0_Wg, dec0_bg, dec0_Wc, dec0_bc, dec1_Wg, dec1_bg, dec1_Wc, dec1_bc, proj_W, proj_b, gru_Wih, gru_Whh, gru_bih, gru_bhh, Wkey, Wquery):
    raise NotImplementedError("write your pallas kernel here")


# speedup vs baseline: 7.4791x; 7.4791x over previous
import jax
import jax.numpy as jnp
from jax import lax
from jax.experimental import pallas as pl

_N = 207
_NP = 208
_HID = 64
_IN = 2
_OUT = 1
_T = 12
_B = 16
_NM = 3
_TOPK = 30
_BNP = _B * _NP


def _body(xt_ref, wih_ref, whh_ref, bih_ref, bhh_ref, wkey_ref, wq_ref,
          e0gx_ref, e0gh_ref, e0bg_ref, e0cx_ref, e0ch_ref, e0bc_ref,
          e1gx_ref, e1gh_ref, e1bg_ref, e1cx_ref, e1ch_ref, e1bc_ref,
          d0gx_ref, d0gh_ref, d0bg_ref, d0cx_ref, d0ch_ref, d0bc_ref,
          d1gx_ref, d1gh_ref, d1bg_ref, d1cx_ref, d1ch_ref, d1bc_ref,
          projw_ref, projb_ref, out_ref):
    f32 = jnp.float32
    sig = jax.nn.sigmoid

    wih = wih_ref[...]
    whh = whh_ref[...]
    bih = bih_ref[...]
    bhh = bhh_ref[...]

    def gru_step(t, h):
        xtt = xt_ref[pl.ds(t, 1)].reshape(_BNP, _IN)
        gi = jnp.dot(xtt, wih, preferred_element_type=f32) + bih
        gh = jnp.dot(h, whh, preferred_element_type=f32) + bhh
        r = sig(gi[:, :_HID] + gh[:, :_HID])
        z = sig(gi[:, _HID:2 * _HID] + gh[:, _HID:2 * _HID])
        n = jnp.tanh(gi[:, 2 * _HID:] + r * gh[:, 2 * _HID:])
        return (1.0 - z) * n + z * h

    h = lax.fori_loop(0, _T, gru_step, jnp.zeros((_BNP, _HID), f32))

    keyv = jnp.dot(h, wkey_ref[...], preferred_element_type=f32)
    qryv = jnp.dot(h, wq_ref[...], preferred_element_type=f32)
    col = lax.broadcasted_iota(jnp.int32, (_NP, _NP), 1)
    colmask = col < _N
    acc = jnp.zeros((_NP, _NP), f32)
    for b in range(_B):
        kb = keyv[b * _NP:(b + 1) * _NP, :]
        qb = qryv[b * _NP:(b + 1) * _NP, :]
        ab = lax.dot_general(kb, qb, (((1,), (1,)), ((), ())),
                             preferred_element_type=f32)
        ab = jnp.maximum(ab, 0.0)
        ab = jnp.where(colmask, ab, -1e30)
        ab = ab - jnp.max(ab, axis=1, keepdims=True)
        e = jnp.where(colmask, jnp.exp(ab), 0.0)
        acc = acc + e / jnp.sum(e, axis=1, keepdims=True)
    adj = acc * (1.0 / _B)

    work = jnp.where(colmask, adj, -1.0)
    thresh = None
    for _ in range(_TOPK):
        thresh = jnp.max(work, axis=1, keepdims=True)
        ismax = work == thresh
        pos = jnp.min(jnp.where(ismax, col, _NP), axis=1, keepdims=True)
        work = jnp.where(col == pos, -1.0, work)
    rowmask = lax.broadcasted_iota(jnp.int32, (_NP, _NP), 0) < _N
    adj_k = jnp.where((adj >= thresh) & colmask & rowmask, adj, 0.0)
    d = jnp.sum(adj_k, axis=1, keepdims=True)
    dinv = jnp.where(d > 0.0, 1.0 / d, 0.0)
    support = dinv * adj_k

    def smul(xv):
        parts = [jnp.dot(support, xv[b * _NP:(b + 1) * _NP, :],
                         preferred_element_type=f32) for b in range(_B)]
        return jnp.concatenate(parts, axis=0)

    def matx(xv, w):
        if w.shape[0] <= 2:
            r = xv[:, 0:1] * w[0:1, :]
            for ci in range(1, w.shape[0]):
                r = r + xv[:, ci:ci + 1] * w[ci:ci + 1, :]
            return r
        return jnp.dot(xv, w, preferred_element_type=f32)

    def cell(xin, hin, wgx, wgh, bg, wcx, wch, bc):
        sx1 = smul(xin)
        sx2 = 2.0 * smul(sx1) - xin
        sh1 = smul(hin)
        sh2 = 2.0 * smul(sh1) - hin
        g = sig(matx(xin, wgx[0]) + matx(sx1, wgx[1]) + matx(sx2, wgx[2])
                + jnp.dot(hin, wgh[0], preferred_element_type=f32)
                + jnp.dot(sh1, wgh[1], preferred_element_type=f32)
                + jnp.dot(sh2, wgh[2], preferred_element_type=f32) + bg)
        r = g[:, :_HID]
        u = g[:, _HID:]
        rh = r * hin
        t1 = smul(rh)
        t2 = 2.0 * smul(t1) - rh
        cand = jnp.tanh(matx(xin, wcx[0]) + matx(sx1, wcx[1]) + matx(sx2, wcx[2])
                        + jnp.dot(rh, wch[0], preferred_element_type=f32)
                        + jnp.dot(t1, wch[1], preferred_element_type=f32)
                        + jnp.dot(t2, wch[2], preferred_element_type=f32) + bc)
        return u * hin + (1.0 - u) * cand

    e0gx = e0gx_ref[...]; e0gh = e0gh_ref[...]; e0bg = e0bg_ref[...]
    e0cx = e0cx_ref[...]; e0ch = e0ch_ref[...]; e0bc = e0bc_ref[...]
    e1gx = e1gx_ref[...]; e1gh = e1gh_ref[...]; e1bg = e1bg_ref[...]
    e1cx = e1cx_ref[...]; e1ch = e1ch_ref[...]; e1bc = e1bc_ref[...]
    d0gx = d0gx_ref[...]; d0gh = d0gh_ref[...]; d0bg = d0bg_ref[...]
    d0cx = d0cx_ref[...]; d0ch = d0ch_ref[...]; d0bc = d0bc_ref[...]
    d1gx = d1gx_ref[...]; d1gh = d1gh_ref[...]; d1bg = d1bg_ref[...]
    d1cx = d1cx_ref[...]; d1ch = d1ch_ref[...]; d1bc = d1bc_ref[...]
    projw = projw_ref[...]
    projb = projb_ref[...]

    def enc_step(t, hh):
        h0, h1 = hh
        x_t = xt_ref[pl.ds(t, 1)].reshape(_BNP, _IN)
        h0 = cell(x_t, h0, e0gx, e0gh, e0bg, e0cx, e0ch, e0bc)
        h1 = cell(h0, h1, e1gx, e1gh, e1bg, e1cx, e1ch, e1bc)
        return (h0, h1)

    zst = jnp.zeros((_BNP, _HID), f32)
    h0, h1 = lax.fori_loop(0, _T, enc_step, (zst, zst))

    tcol = lax.broadcasted_iota(jnp.int32, (_BNP, _T), 1)

    def dec_step(t, carry):
        g0, g1, cur, outacc = carry
        g0 = cell(cur, g0, d0gx, d0gh, d0bg, d0cx, d0ch, d0bc)
        g1 = cell(g0, g1, d1gx, d1gh, d1bg, d1cx, d1ch, d1bc)
        p = jnp.sum(g1 * projw, axis=1, keepdims=True) + projb
        outacc = jnp.where(tcol == t, p, outacc)
        return (g0, g1, p, outacc)

    _, _, _, outacc = lax.fori_loop(
        0, _T, dec_step,
        (h0, h1, jnp.zeros((_BNP, _OUT), f32), jnp.zeros((_BNP, _T), f32)))
    out_ref[...] = outacc


def kernel(x, enc0_Wg, enc0_bg, enc0_Wc, enc0_bc, enc1_Wg, enc1_bg, enc1_Wc, enc1_bc,
           dec0_Wg, dec0_bg, dec0_Wc, dec0_bc, dec1_Wg, dec1_bg, dec1_Wc, dec1_bc,
           proj_W, proj_b, gru_Wih, gru_Whh, gru_bih, gru_bhh, Wkey, Wquery):
    f32 = jnp.float32
    xt = jnp.transpose(x, (1, 0, 2, 3))
    xt = jnp.pad(xt, ((0, 0), (0, 0), (0, _NP - _N), (0, 0)))
    xt = xt.reshape(_T, _BNP, _IN)

    def deint(w, cin):
        s = jnp.stack([w[m::_NM] for m in range(_NM)])
        return s[:, :cin, :], s[:, cin:, :]

    e0gx, e0gh = deint(enc0_Wg, _IN); e0cx, e0ch = deint(enc0_Wc, _IN)
    e1gx, e1gh = deint(enc1_Wg, _HID); e1cx, e1ch = deint(enc1_Wc, _HID)
    d0gx, d0gh = deint(dec0_Wg, _OUT); d0cx, d0ch = deint(dec0_Wc, _OUT)
    d1gx, d1gh = deint(dec1_Wg, _HID); d1cx, d1ch = deint(dec1_Wc, _HID)

    args = (
        xt,
        gru_Wih.T.astype(f32), gru_Whh.T.astype(f32),
        gru_bih[None], gru_bhh[None],
        Wkey, Wquery,
        e0gx, e0gh, enc0_bg[None], e0cx, e0ch, enc0_bc[None],
        e1gx, e1gh, enc1_bg[None], e1cx, e1ch, enc1_bc[None],
        d0gx, d0gh, dec0_bg[None], d0cx, d0ch, dec0_bc[None],
        d1gx, d1gh, dec1_bg[None], d1cx, d1ch, dec1_bc[None],
        proj_W.T, proj_b.reshape(1, 1),
    )

    out = pl.pallas_call(
        _body,
        out_shape=jax.ShapeDtypeStruct((_BNP, _T), f32),
    )(*args)
    out = out.reshape(_B, _NP, _T)
    return jnp.transpose(out, (0, 2, 1))[:, :, :_N, None]
